# baseline (speedup 1.0000x reference)
import jax
import jax.numpy as jnp
from jax import lax
from jax.experimental import pallas as pl
from jax.experimental.pallas import tpu as pltpu
from jax.experimental.pallas import tpu_sc as plsc

B, S, D, L = 1024, 512, 256, 50
NC, NS, LANES = 2, 16, 16
NW = NC * NS
RPW = B // NW
GROUPS = RPW // LANES
DV = D // LANES


def _sc_body(cn_hbm, enc_hbm, mask_hbm, sib_hbm, out_hbm,
             mask_v, sib_v, cn_v, rows_v, idx_v, mfac_v,
             sem_in, sem_cn, sem_g0, sem_g1, sem_out):
    wid = lax.axis_index("c") * NS + lax.axis_index("s")
    base = wid * RPW

    c_mask = pltpu.async_copy(mask_hbm.at[pl.ds(base, RPW)], mask_v, sem_in)
    c_sib = pltpu.async_copy(sib_hbm.at[pl.ds(base, RPW)], sib_v, sem_in)
    c_cn = pltpu.async_copy(cn_hbm.at[pl.ds(base, RPW)], cn_v, sem_cn)
    c_mask.wait()
    c_sib.wait()

    lane = lax.iota(jnp.int32, LANES)
    sem_gs = [sem_g0, sem_g1]
    gathers = []
    for g in range(GROUPS):
        sl = pl.ds(g * LANES, LANES)
        j = lane + g * LANES
        n = plsc.load_gather(mask_v, [j, jnp.zeros((LANES,), jnp.int32)])
        for l in range(1, L):
            n = n + plsc.load_gather(mask_v, [j, jnp.full((LANES,), l, jnp.int32)])
        last = jnp.clip(n - 1, 0, L - 1)
        sib = plsc.load_gather(sib_v, [j, last])
        idx_v[sl] = (base + j) * S + sib
        mfac_v[sl] = (n != 0).astype(jnp.float32)
        gathers.append(pltpu.async_copy(
            enc_hbm.at[idx_v.at[sl]], rows_v.at[sl], sem_gs[g]))

    c_cn.wait()
    outs = []
    for g in range(GROUPS):
        gathers[g].wait()
        for j in range(LANES):
            r = g * LANES + j
            mrow = plsc.load_gather(mfac_v, [jnp.full((LANES,), r, jnp.int32)])
            for d in range(DV):
                dsl = pl.ds(d * LANES, LANES)
                rows_v[r, dsl] = cn_v[r, dsl] + mrow * rows_v[r, dsl]
        sl = pl.ds(g * LANES, LANES)
        outs.append(pltpu.async_copy(
            rows_v.at[sl], out_hbm.at[pl.ds(base + g * LANES, LANES)], sem_out))
    for c in outs:
        c.wait()


def kernel(current_node, encoded_input, siblings, siblings_mask):
    enc2 = encoded_input.reshape(B * S, D)
    mask_w = siblings_mask.astype(jnp.int32)
    sib_w = siblings.astype(jnp.int32)

    run = pl.kernel(
        _sc_body,
        out_type=jax.ShapeDtypeStruct((B, D), jnp.float32),
        mesh=plsc.VectorSubcoreMesh(core_axis_name="c", subcore_axis_name="s"),
        compiler_params=pltpu.CompilerParams(needs_layout_passes=False),
        scratch_types=[
            pltpu.VMEM((RPW, L), jnp.int32),
            pltpu.VMEM((RPW, L), jnp.int32),
            pltpu.VMEM((RPW, D), jnp.float32),
            pltpu.VMEM((RPW, D), jnp.float32),
            pltpu.VMEM((RPW,), jnp.int32),
            pltpu.VMEM((RPW,), jnp.float32),
            pltpu.SemaphoreType.DMA,
            pltpu.SemaphoreType.DMA,
            pltpu.SemaphoreType.DMA,
            pltpu.SemaphoreType.DMA,
            pltpu.SemaphoreType.DMA,
        ],
    )
    return run(current_node, enc2, mask_w, sib_w)


# speedup vs baseline: 1.0049x; 1.0049x over previous
import jax
import jax.numpy as jnp
from jax import lax
from jax.experimental import pallas as pl
from jax.experimental.pallas import tpu as pltpu
from jax.experimental.pallas import tpu_sc as plsc

B, S, D, L = 1024, 512, 256, 50
NC, NS, LANES = 2, 16, 16
NW = NC * NS
RPW = B // NW
GROUPS = RPW // LANES
DV = D // LANES


def _sc_body(cn_hbm, enc_hbm, mask_hbm, sib_hbm, out_hbm,
             mask_v, sib_v, cn_v, rows_v, idx_v, mfac_v,
             sem_in0, sem_in1, sem_cn0, sem_cn1, sem_g0, sem_g1, sem_out):
    wid = lax.axis_index("c") * NS + lax.axis_index("s")
    base = wid * RPW
    half = LANES * L

    sem_ins = [sem_in0, sem_in1]
    sem_cns = [sem_cn0, sem_cn1]
    sem_gs = [sem_g0, sem_g1]
    c_ins, c_cns = [], []
    for g in range(GROUPS):
        hsl = pl.ds(g * half, half)
        src = pl.ds(pl.multiple_of(wid * (RPW * L) + g * half, 8), half)
        c_ins.append((
            pltpu.async_copy(mask_hbm.at[src], mask_v.at[hsl], sem_ins[g]),
            pltpu.async_copy(sib_hbm.at[src], sib_v.at[hsl], sem_ins[g]),
        ))
        rsl = pl.ds(g * LANES, LANES)
        c_cns.append(pltpu.async_copy(
            cn_hbm.at[pl.ds(base + g * LANES, LANES)], cn_v.at[rsl], sem_cns[g]))

    lane = lax.iota(jnp.int32, LANES)
    gathers = []
    for g in range(GROUPS):
        c_ins[g][0].wait()
        c_ins[g][1].wait()
        sl = pl.ds(g * LANES, LANES)
        j = lane + g * LANES
        rowbase = j * L
        n = plsc.load_gather(mask_v, [rowbase])
        for l in range(1, L):
            n = n + plsc.load_gather(mask_v, [rowbase + l])
        last = jnp.clip(n - 1, 0, L - 1)
        sib = plsc.load_gather(sib_v, [rowbase + last])
        idx_v[sl] = (base + j) * S + sib
        mfac_v[sl] = (n != 0).astype(jnp.float32)
        gathers.append(pltpu.async_copy(
            enc_hbm.at[idx_v.at[sl]], rows_v.at[sl], sem_gs[g]))

    outs = []
    for g in range(GROUPS):
        c_cns[g].wait()
        gathers[g].wait()
        for j in range(LANES):
            r = g * LANES + j
            mrow = plsc.load_gather(mfac_v, [jnp.full((LANES,), r, jnp.int32)])
            for d in range(DV):
                dsl = pl.ds(d * LANES, LANES)
                rows_v[r, dsl] = cn_v[r, dsl] + mrow * rows_v[r, dsl]
        sl = pl.ds(g * LANES, LANES)
        outs.append(pltpu.async_copy(
            rows_v.at[sl], out_hbm.at[pl.ds(base + g * LANES, LANES)], sem_out))
    for c in outs:
        c.wait()


def kernel(current_node, encoded_input, siblings, siblings_mask):
    enc2 = encoded_input.reshape(B * S, D)
    mask_w = siblings_mask.astype(jnp.int32).reshape(B * L)
    sib_w = siblings.astype(jnp.int32).reshape(B * L)

    run = pl.kernel(
        _sc_body,
        out_type=jax.ShapeDtypeStruct((B, D), jnp.float32),
        mesh=plsc.VectorSubcoreMesh(core_axis_name="c", subcore_axis_name="s"),
        compiler_params=pltpu.CompilerParams(needs_layout_passes=False),
        scratch_types=[
            pltpu.VMEM((RPW * L,), jnp.int32),
            pltpu.VMEM((RPW * L,), jnp.int32),
            pltpu.VMEM((RPW, D), jnp.float32),
            pltpu.VMEM((RPW, D), jnp.float32),
            pltpu.VMEM((RPW,), jnp.int32),
            pltpu.VMEM((RPW,), jnp.float32),
            pltpu.SemaphoreType.DMA,
            pltpu.SemaphoreType.DMA,
            pltpu.SemaphoreType.DMA,
            pltpu.SemaphoreType.DMA,
            pltpu.SemaphoreType.DMA,
            pltpu.SemaphoreType.DMA,
            pltpu.SemaphoreType.DMA,
        ],
    )
    return run(current_node, enc2, mask_w, sib_w)


# speedup vs baseline: 1.0063x; 1.0014x over previous
import jax
import jax.numpy as jnp
from jax import lax
from jax.experimental import pallas as pl
from jax.experimental.pallas import tpu as pltpu
from jax.experimental.pallas import tpu_sc as plsc

B, S, D, L = 1024, 512, 256, 50
NC, NS, LANES = 2, 16, 16
NW = NC * NS
RPW = B // NW
GROUPS = RPW // LANES
DV = D // LANES


def _sc_body(cn_hbm, enc_hbm, mask_hbm, sib_hbm, out_hbm,
             mask_v, sib_v, cn_v, rows_v, idx_v, mfac_v,
             sem_in, sem_cn, sem_g0, sem_g1, sem_out):
    wid = lax.axis_index("c") * NS + lax.axis_index("s")
    base = wid * RPW

    c_mask = pltpu.async_copy(mask_hbm.at[pl.ds(base, RPW)], mask_v, sem_in)
    c_sib = pltpu.async_copy(sib_hbm.at[pl.ds(base, RPW)], sib_v, sem_in)
    c_cn = pltpu.async_copy(cn_hbm.at[pl.ds(base, RPW)], cn_v, sem_cn)
    c_mask.wait()
    c_sib.wait()

    lane = lax.iota(jnp.int32, LANES)
    sem_gs = [sem_g0, sem_g1]
    gathers = []
    for g in range(GROUPS):
        sl = pl.ds(g * LANES, LANES)
        j = lane + g * LANES
        n = plsc.load_gather(mask_v, [j, jnp.zeros((LANES,), jnp.int32)])
        for l in range(1, L):
            n = n + plsc.load_gather(mask_v, [j, jnp.full((LANES,), l, jnp.int32)])
        last = jnp.clip(n - 1, 0, L - 1)
        sib = plsc.load_gather(sib_v, [j, last])
        idx_v[sl] = (base + j) * S + sib
        mfac_v[sl] = (n != 0).astype(jnp.float32)
        gathers.append(pltpu.async_copy(
            enc_hbm.at[idx_v.at[sl]], rows_v.at[sl], sem_gs[g]))

    c_cn.wait()
    outs = []
    for g in range(GROUPS):
        gathers[g].wait()
        for j in range(LANES):
            r = g * LANES + j
            mrow = plsc.load_gather(mfac_v, [jnp.full((LANES,), r, jnp.int32)])
            for d in range(DV):
                dsl = pl.ds(d * LANES, LANES)
                rows_v[r, dsl] = cn_v[r, dsl] + mrow * rows_v[r, dsl]
        sl = pl.ds(g * LANES, LANES)
        outs.append(pltpu.async_copy(
            rows_v.at[sl], out_hbm.at[pl.ds(base + g * LANES, LANES)], sem_out))
    for c in outs:
        c.wait()


def kernel(current_node, encoded_input, siblings, siblings_mask):
    enc2 = encoded_input.reshape(B * S, D)
    mask_w = siblings_mask.astype(jnp.int32)
    sib_w = siblings.astype(jnp.int32)

    run = pl.kernel(
        _sc_body,
        out_type=jax.ShapeDtypeStruct((B, D), jnp.float32),
        mesh=plsc.VectorSubcoreMesh(core_axis_name="c", subcore_axis_name="s"),
        compiler_params=pltpu.CompilerParams(
            needs_layout_passes=False, use_tc_tiling_on_sc=True),
        scratch_types=[
            pltpu.VMEM((RPW, L), jnp.int32),
            pltpu.VMEM((RPW, L), jnp.int32),
            pltpu.VMEM((RPW, D), jnp.float32),
            pltpu.VMEM((RPW, D), jnp.float32),
            pltpu.VMEM((RPW,), jnp.int32),
            pltpu.VMEM((RPW,), jnp.float32),
            pltpu.SemaphoreType.DMA,
            pltpu.SemaphoreType.DMA,
            pltpu.SemaphoreType.DMA,
            pltpu.SemaphoreType.DMA,
            pltpu.SemaphoreType.DMA,
        ],
    )
    return run(current_node, enc2, mask_w, sib_w)


# speedup vs baseline: 1.0401x; 1.0336x over previous
import jax
import jax.numpy as jnp
from jax import lax
from jax.experimental import pallas as pl
from jax.experimental.pallas import tpu as pltpu
from jax.experimental.pallas import tpu_sc as plsc

B, S, D, L = 1024, 512, 256, 50
NC, NS, LANES = 2, 16, 16
NW = NC * NS
RPW = B // NW
GROUPS = RPW // LANES
DV = D // LANES


def _sc_body(cn_hbm, enc_hbm, mask_hbm, sib_hbm, out_hbm,
             mask_v, sib_v, cn_v, rows_v, idx_v, mfac_v,
             sem_in, sem_cn, sem_g0, sem_g1, sem_out):
    wid = lax.axis_index("c") * NS + lax.axis_index("s")
    base = wid * RPW

    c_mask = pltpu.async_copy(mask_hbm.at[pl.ds(base, RPW)], mask_v, sem_in)
    c_sib = pltpu.async_copy(sib_hbm.at[pl.ds(base, RPW)], sib_v, sem_in)
    c_cn = pltpu.async_copy(cn_hbm.at[pl.ds(base, RPW)], cn_v, sem_cn)
    c_mask.wait()
    c_sib.wait()

    lane = lax.iota(jnp.int32, LANES)
    sem_gs = [sem_g0, sem_g1]
    gathers = []
    for g in range(GROUPS):
        sl = pl.ds(g * LANES, LANES)
        j = lane + g * LANES

        def msum(l, acc):
            return acc + plsc.load_gather(
                mask_v, [j, jnp.full((LANES,), l, jnp.int32)])

        n = lax.fori_loop(0, L, msum, jnp.zeros((LANES,), jnp.int32))
        last = jnp.clip(n - 1, 0, L - 1)
        sib = plsc.load_gather(sib_v, [j, last])
        idx_v[sl] = (base + j) * S + sib
        mfac_v[sl] = (n != 0).astype(jnp.float32)
        gathers.append(pltpu.async_copy(
            enc_hbm.at[idx_v.at[sl]], rows_v.at[sl], sem_gs[g]))

    c_cn.wait()
    outs = []
    for g in range(GROUPS):
        gathers[g].wait()

        def combine(r, carry):
            mrow = plsc.load_gather(mfac_v, [jnp.full((LANES,), r, jnp.int32)])
            for d in range(DV):
                dsl = pl.ds(d * LANES, LANES)
                rows_v[r, dsl] = cn_v[r, dsl] + mrow * rows_v[r, dsl]
            return carry

        lax.fori_loop(g * LANES, (g + 1) * LANES, combine, 0)
        sl = pl.ds(g * LANES, LANES)
        outs.append(pltpu.async_copy(
            rows_v.at[sl], out_hbm.at[pl.ds(base + g * LANES, LANES)], sem_out))
    for c in outs:
        c.wait()


def kernel(current_node, encoded_input, siblings, siblings_mask):
    enc2 = encoded_input.reshape(B * S, D)
    mask_w = siblings_mask.astype(jnp.int32)
    sib_w = siblings.astype(jnp.int32)

    run = pl.kernel(
        _sc_body,
        out_type=jax.ShapeDtypeStruct((B, D), jnp.float32),
        mesh=plsc.VectorSubcoreMesh(core_axis_name="c", subcore_axis_name="s"),
        compiler_params=pltpu.CompilerParams(
            needs_layout_passes=False, use_tc_tiling_on_sc=True),
        scratch_types=[
            pltpu.VMEM((RPW, L), jnp.int32),
            pltpu.VMEM((RPW, L), jnp.int32),
            pltpu.VMEM((RPW, D), jnp.float32),
            pltpu.VMEM((RPW, D), jnp.float32),
            pltpu.VMEM((RPW,), jnp.int32),
            pltpu.VMEM((RPW,), jnp.float32),
            pltpu.SemaphoreType.DMA,
            pltpu.SemaphoreType.DMA,
            pltpu.SemaphoreType.DMA,
            pltpu.SemaphoreType.DMA,
            pltpu.SemaphoreType.DMA,
        ],
    )
    return run(current_node, enc2, mask_w, sib_w)


# speedup vs baseline: 1.0937x; 1.0515x over previous
import jax
import jax.numpy as jnp
from jax import lax
from jax.experimental import pallas as pl
from jax.experimental.pallas import tpu as pltpu
from jax.experimental.pallas import tpu_sc as plsc

B, S, D, L = 1024, 512, 256, 50
NC, NS, LANES = 2, 16, 16
NW = NC * NS
RPW = B // NW
GROUPS = RPW // LANES
DV = D // LANES


def _sc_body(cn_hbm, enc_hbm, mask_hbm, sib_hbm, out_hbm,
             mask_v, sib_v, cn_v, rows_v, idx_v, mfac_v,
             sem_in, sem_cn, sem_g0, sem_g1, sem_out):
    wid = lax.axis_index("c") * NS + lax.axis_index("s")
    base = wid * RPW

    cblock = pl.multiple_of(wid // 4 * 128, 128)
    ccol = wid % 4 * 128 // 4
    csl = pl.ds(cblock, 128)
    c_mask = pltpu.async_copy(mask_hbm.at[:, csl], mask_v, sem_in)
    c_sib = pltpu.async_copy(sib_hbm.at[:, csl], sib_v, sem_in)
    c_cn = pltpu.async_copy(cn_hbm.at[pl.ds(base, RPW)], cn_v, sem_cn)
    c_mask.wait()
    c_sib.wait()

    lane = lax.iota(jnp.int32, LANES)
    sem_gs = [sem_g0, sem_g1]
    gathers = []
    for g in range(GROUPS):
        sl = pl.ds(g * LANES, LANES)
        j = lane + g * LANES
        cols = ccol + j

        def msum(l, acc):
            return acc + plsc.load_gather(
                mask_v, [jnp.full((LANES,), l, jnp.int32), cols])

        n = lax.fori_loop(0, L, msum, jnp.zeros((LANES,), jnp.int32))
        last = jnp.clip(n - 1, 0, L - 1)
        sib = plsc.load_gather(sib_v, [last, cols])
        idx_v[sl] = (base + j) * S + sib
        mfac_v[sl] = (n != 0).astype(jnp.float32)
        gathers.append(pltpu.async_copy(
            enc_hbm.at[idx_v.at[sl]], rows_v.at[sl], sem_gs[g]))

    c_cn.wait()
    outs = []
    for g in range(GROUPS):
        gathers[g].wait()

        def combine(r, carry):
            mrow = plsc.load_gather(mfac_v, [jnp.full((LANES,), r, jnp.int32)])
            for d in range(DV):
                dsl = pl.ds(d * LANES, LANES)
                rows_v[r, dsl] = cn_v[r, dsl] + mrow * rows_v[r, dsl]
            return carry

        lax.fori_loop(g * LANES, (g + 1) * LANES, combine, 0)
        sl = pl.ds(g * LANES, LANES)
        outs.append(pltpu.async_copy(
            rows_v.at[sl], out_hbm.at[pl.ds(base + g * LANES, LANES)], sem_out))
    for c in outs:
        c.wait()


def kernel(current_node, encoded_input, siblings, siblings_mask):
    enc2 = encoded_input.reshape(B * S, D)
    mask_w = siblings_mask.T.astype(jnp.int32)
    sib_w = siblings.astype(jnp.int32).T

    run = pl.kernel(
        _sc_body,
        out_type=jax.ShapeDtypeStruct((B, D), jnp.float32),
        mesh=plsc.VectorSubcoreMesh(core_axis_name="c", subcore_axis_name="s"),
        compiler_params=pltpu.CompilerParams(
            needs_layout_passes=False, use_tc_tiling_on_sc=True),
        scratch_types=[
            pltpu.VMEM((L, 128), jnp.int32),
            pltpu.VMEM((L, 128), jnp.int32),
            pltpu.VMEM((RPW, D), jnp.float32),
            pltpu.VMEM((RPW, D), jnp.float32),
            pltpu.VMEM((RPW,), jnp.int32),
            pltpu.VMEM((RPW,), jnp.float32),
            pltpu.SemaphoreType.DMA,
            pltpu.SemaphoreType.DMA,
            pltpu.SemaphoreType.DMA,
            pltpu.SemaphoreType.DMA,
            pltpu.SemaphoreType.DMA,
        ],
    )
    return run(current_node, enc2, mask_w, sib_w)


# speedup vs baseline: 1.0995x; 1.0053x over previous
import jax
import jax.numpy as jnp
from jax import lax
from jax.experimental import pallas as pl
from jax.experimental.pallas import tpu as pltpu
from jax.experimental.pallas import tpu_sc as plsc

B, S, D, L = 1024, 512, 256, 50
NC, NS, LANES = 2, 16, 16
NW = NC * NS
RPW = B // NW
GROUPS = RPW // LANES
DV = D // LANES


def _sc_body(cn_hbm, enc_hbm, mask_hbm, sib_hbm, out_hbm,
             mask_v, sib_v, cn_v, rows_v, idx_v, mfac_v,
             sem_in, sem_cn, sem_g0, sem_g1, sem_out):
    wid = lax.axis_index("c") * NS + lax.axis_index("s")
    base = wid * RPW

    cblock = pl.multiple_of(wid // 4 * 128, 128)
    ccol = wid % 4 * 128 // 4
    csl = pl.ds(cblock, 128)
    c_mask = pltpu.async_copy(mask_hbm.at[:, csl], mask_v, sem_in)
    c_sib = pltpu.async_copy(sib_hbm.at[:, csl], sib_v, sem_in)
    c_cn = pltpu.async_copy(cn_hbm.at[pl.ds(base, RPW)], cn_v, sem_cn)
    c_mask.wait()
    c_sib.wait()

    lane = lax.iota(jnp.int32, LANES)
    sem_gs = [sem_g0, sem_g1]
    gathers = []
    for g in range(GROUPS):
        sl = pl.ds(g * LANES, LANES)
        j = lane + g * LANES
        cols = ccol + j

        def msum(i, acc):
            l0 = i * 5
            for k in range(5):
                acc = acc + plsc.load_gather(
                    mask_v, [jnp.full((LANES,), l0 + k, jnp.int32), cols])
            return acc

        n = lax.fori_loop(0, L // 5, msum, jnp.zeros((LANES,), jnp.int32))
        last = jnp.clip(n - 1, 0, L - 1)
        sib = plsc.load_gather(sib_v, [last, cols])
        idx_v[sl] = (base + j) * S + sib
        mfac_v[sl] = (n != 0).astype(jnp.float32)
        gathers.append(pltpu.async_copy(
            enc_hbm.at[idx_v.at[sl]], rows_v.at[sl], sem_gs[g]))

    c_cn.wait()
    outs = []
    for g in range(GROUPS):
        gathers[g].wait()

        def combine(r, carry):
            mrow = plsc.load_gather(mfac_v, [jnp.full((LANES,), r, jnp.int32)])
            for d in range(DV):
                dsl = pl.ds(d * LANES, LANES)
                rows_v[r, dsl] = cn_v[r, dsl] + mrow * rows_v[r, dsl]
            return carry

        lax.fori_loop(g * LANES, (g + 1) * LANES, combine, 0)
        sl = pl.ds(g * LANES, LANES)
        outs.append(pltpu.async_copy(
            rows_v.at[sl], out_hbm.at[pl.ds(base + g * LANES, LANES)], sem_out))
    for c in outs:
        c.wait()


def kernel(current_node, encoded_input, siblings, siblings_mask):
    enc2 = encoded_input.reshape(B * S, D)
    mask_w = siblings_mask.T.astype(jnp.int32)
    sib_w = siblings.astype(jnp.int32).T

    run = pl.kernel(
        _sc_body,
        out_type=jax.ShapeDtypeStruct((B, D), jnp.float32),
        mesh=plsc.VectorSubcoreMesh(core_axis_name="c", subcore_axis_name="s"),
        compiler_params=pltpu.CompilerParams(
            needs_layout_passes=False, use_tc_tiling_on_sc=True),
        scratch_types=[
            pltpu.VMEM((L, 128), jnp.int32),
            pltpu.VMEM((L, 128), jnp.int32),
            pltpu.VMEM((RPW, D), jnp.float32),
            pltpu.VMEM((RPW, D), jnp.float32),
            pltpu.VMEM((RPW,), jnp.int32),
            pltpu.VMEM((RPW,), jnp.float32),
            pltpu.SemaphoreType.DMA,
            pltpu.SemaphoreType.DMA,
            pltpu.SemaphoreType.DMA,
            pltpu.SemaphoreType.DMA,
            pltpu.SemaphoreType.DMA,
        ],
    )
    return run(current_node, enc2, mask_w, sib_w)
